# Initial kernel scaffold; baseline (speedup 1.0000x reference)
import jax
import jax.numpy as jnp
from jax.experimental import pallas as pl


def kernel(x, edge_index, w_init, w_root, biases, fc_w, fc_b):
    raise NotImplementedError("write your pallas kernel here")


# speedup vs baseline: 2.9115x; 2.9115x over previous
import functools

import jax
import jax.numpy as jnp
from jax import lax
from jax.experimental import pallas as pl
from jax.experimental.pallas import tpu as pltpu
from jax.experimental.pallas import tpu_sc as plsc

N = 10000
NPAD = 10240
E = 160000
CHUNKS = 4
CW = 128
NTILES = 16
NPT = NPAD // NTILES
K = 128
EPAD = 163840
NB = EPAD // (NTILES * K)
RB = 1000


def _t_chunks_body(deg_ref, h_ref, w_ref, out_ref):
    deg = deg_ref[...]
    dis = jnp.where(deg > 0.0, lax.rsqrt(jnp.where(deg > 0.0, deg, 1.0)), 0.0)
    t = jnp.dot(h_ref[...], w_ref[...], preferred_element_type=jnp.float32)
    t = t * dis
    for cc in range(CHUNKS):
        out_ref[cc] = t[:, cc * CW:(cc + 1) * CW]


def _t_chunks(deg, h, w):
    n, d_in = h.shape
    return pl.pallas_call(
        _t_chunks_body,
        grid=(n // RB,),
        in_specs=[
            pl.BlockSpec((RB, 1), lambda r: (r, 0)),
            pl.BlockSpec((RB, d_in), lambda r: (r, 0)),
            pl.BlockSpec((d_in, CHUNKS * CW), lambda r: (0, 0)),
        ],
        out_specs=pl.BlockSpec((CHUNKS, RB, CW), lambda r: (0, r, 0)),
        out_shape=jax.ShapeDtypeStruct((CHUNKS, NPAD, CW), jnp.float32),
    )(deg, h, w)


def _combine_body(deg_ref, agg_ref, h_ref, w_ref, b_ref, out_ref):
    deg = deg_ref[...]
    dis = jnp.where(deg > 0.0, lax.rsqrt(jnp.where(deg > 0.0, deg, 1.0)), 0.0)
    agg = jnp.concatenate([agg_ref[cc] for cc in range(CHUNKS)], axis=-1)
    r = jnp.dot(h_ref[...], w_ref[...], preferred_element_type=jnp.float32)
    out_ref[...] = jnp.maximum(agg * dis + r + b_ref[...], 0.0)


def _combine(deg, agg, h, w, b):
    n, d_in = h.shape
    return pl.pallas_call(
        _combine_body,
        grid=(n // RB,),
        in_specs=[
            pl.BlockSpec((RB, 1), lambda r: (r, 0)),
            pl.BlockSpec((CHUNKS, RB, CW), lambda r: (0, r, 0)),
            pl.BlockSpec((RB, d_in), lambda r: (r, 0)),
            pl.BlockSpec((d_in, CHUNKS * CW), lambda r: (0, 0)),
            pl.BlockSpec((1, CHUNKS * CW), lambda r: (0, 0)),
        ],
        out_specs=pl.BlockSpec((RB, CHUNKS * CW), lambda r: (r, 0)),
        out_shape=jax.ShapeDtypeStruct((n, CHUNKS * CW), jnp.float32),
    )(deg, agg, h, w, b)


def _fc_body(h_ref, w_ref, b_ref, out_ref):
    out_ref[...] = (
        jnp.dot(h_ref[...], w_ref[...], preferred_element_type=jnp.float32)
        + b_ref[...]
    )


def _fc(h2, w, b):
    m, kdim = h2.shape
    d_out = w.shape[1]
    rb = 400
    return pl.pallas_call(
        _fc_body,
        grid=(m // rb,),
        in_specs=[
            pl.BlockSpec((rb, kdim), lambda r: (r, 0)),
            pl.BlockSpec((kdim, d_out), lambda r: (0, 0)),
            pl.BlockSpec((1, d_out), lambda r: (0, 0)),
        ],
        out_specs=pl.BlockSpec((rb, d_out), lambda r: (r, 0)),
        out_shape=jax.ShapeDtypeStruct((m, d_out), jnp.float32),
    )(h2, w, b)


def _sc_degree(col16, zdeg, ones):
    mesh = plsc.VectorSubcoreMesh(core_axis_name="c", subcore_axis_name="s")

    @functools.partial(
        pl.kernel,
        out_type=jax.ShapeDtypeStruct((NPAD, CW), jnp.float32),
        mesh=mesh,
        scratch_types=[
            pltpu.VMEM_SHARED((NPAD, CW), jnp.float32),
            pltpu.VMEM((NB, K), jnp.int32),
            pltpu.VMEM((K, CW), jnp.float32),
        ],
    )
    def k(col_hbm, z_hbm, ones_hbm, out_hbm, accd, colv, onesv):
        c = lax.axis_index("c")
        s = lax.axis_index("s")
        pltpu.sync_copy(col_hbm.at[s], colv)
        pltpu.sync_copy(ones_hbm, onesv)
        pltpu.sync_copy(z_hbm, accd.at[pl.ds(s * NPT, NPT)])
        plsc.subcore_barrier()

        def body(b, carry):
            pltpu.sync_copy(onesv, accd.at[colv.at[b]], add=True)
            return carry

        lax.fori_loop(0, NB, body, 0)
        plsc.subcore_barrier()

        @pl.when(c == 0)
        def _():
            pltpu.sync_copy(accd.at[pl.ds(s * NPT, NPT)],
                            out_hbm.at[pl.ds(s * NPT, NPT)])

    return k(col16, zdeg, ones)


def _sc_edge_pass(t_flat, row16, col16, zrows):
    mesh = plsc.VectorSubcoreMesh(core_axis_name="c", subcore_axis_name="s")

    @functools.partial(
        pl.kernel,
        out_type=jax.ShapeDtypeStruct((CHUNKS * NPAD, CW), jnp.float32),
        mesh=mesh,
        scratch_types=[
            pltpu.VMEM_SHARED((NPAD, CW), jnp.float32),
            pltpu.VMEM((NB, K), jnp.int32),
            pltpu.VMEM((NB, K), jnp.int32),
            pltpu.VMEM((K, CW), jnp.float32),
            pltpu.SemaphoreType.DMA,
        ],
    )
    def k(t_hbm, row_hbm, col_hbm, z_hbm, out_hbm,
          acc, rowv, colv, gbuf, gsem):
        c = lax.axis_index("c")
        s = lax.axis_index("s")
        pltpu.sync_copy(row_hbm.at[s], rowv)
        pltpu.sync_copy(col_hbm.at[s], colv)
        for p in range(2):
            chunk = c * 2 + p
            off = chunk * NPAD
            delta = (c * 2) * NPAD if p == 0 else NPAD
            pltpu.sync_copy(z_hbm, acc.at[pl.ds(s * NPT, NPT)])

            def adj_body(b, carry):
                for j in range(K // 16):
                    sl = pl.ds(j * 16, 16)
                    rowv[b, sl] = rowv[b, sl] + delta
                return carry

            lax.fori_loop(0, NB, adj_body, 0)
            plsc.subcore_barrier()

            def ed_body(b, carry):
                pltpu.async_copy(t_hbm.at[rowv.at[b]], gbuf, gsem).wait()
                pltpu.sync_copy(gbuf, acc.at[colv.at[b]], add=True)
                return carry

            lax.fori_loop(0, NB, ed_body, 0)
            plsc.subcore_barrier()
            pltpu.sync_copy(acc.at[pl.ds(s * NPT, NPT)],
                            out_hbm.at[pl.ds(off + s * NPT, NPT)])
            plsc.subcore_barrier()

    return k(t_flat, row16, col16, zrows)


def kernel(x, edge_index, w_init, w_root, biases, fc_w, fc_b):
    pad = jnp.full((EPAD - edge_index.shape[1],), N, jnp.int32)
    row16 = jnp.concatenate([edge_index[0], pad]).reshape(NTILES, NB, K)
    col16 = jnp.concatenate([edge_index[1], pad]).reshape(NTILES, NB, K)
    zdeg = jnp.zeros((NPT, CW), jnp.float32)
    ones = jnp.ones((K, CW), jnp.float32)
    zrows = jnp.zeros((NPT, CW), jnp.float32)

    deg = _sc_degree(col16, zdeg, ones)[:, :1]

    h = x
    for i in range(len(w_init)):
        t = _t_chunks(deg, h, w_init[i])
        agg = _sc_edge_pass(t.reshape(CHUNKS * NPAD, CW), row16, col16, zrows)
        h = _combine(deg, agg.reshape(CHUNKS, NPAD, CW), h,
                     w_root[i], biases[i].reshape(1, -1))

    h2 = h.reshape(-1, fc_w.shape[0])
    return _fc(h2, fc_w, fc_b.reshape(1, -1))


# speedup vs baseline: 3.2585x; 1.1192x over previous
import functools

import jax
import jax.numpy as jnp
from jax import lax
from jax.experimental import pallas as pl
from jax.experimental.pallas import tpu as pltpu
from jax.experimental.pallas import tpu_sc as plsc

N = 10000
NPAD = 10240
E = 160000
CHUNKS = 4
CW = 128
NTILES = 16
NPT = NPAD // NTILES
K = 128
EPAD = 163840
NB = EPAD // (NTILES * K)
RB = 1000


def _t_chunks_body(deg_ref, h_ref, w_ref, out_ref):
    deg = deg_ref[...]
    dis = jnp.where(deg > 0.0, lax.rsqrt(jnp.where(deg > 0.0, deg, 1.0)), 0.0)
    t = jnp.dot(h_ref[...], w_ref[...], preferred_element_type=jnp.float32)
    t = t * dis
    for cc in range(CHUNKS):
        out_ref[cc] = t[:, cc * CW:(cc + 1) * CW]


def _t_chunks(deg, h, w):
    n, d_in = h.shape
    return pl.pallas_call(
        _t_chunks_body,
        grid=(n // RB,),
        in_specs=[
            pl.BlockSpec((RB, 1), lambda r: (r, 0)),
            pl.BlockSpec((RB, d_in), lambda r: (r, 0)),
            pl.BlockSpec((d_in, CHUNKS * CW), lambda r: (0, 0)),
        ],
        out_specs=pl.BlockSpec((CHUNKS, RB, CW), lambda r: (0, r, 0)),
        out_shape=jax.ShapeDtypeStruct((CHUNKS, NPAD, CW), jnp.float32),
    )(deg, h, w)


def _combine_body(deg_ref, agg_ref, h_ref, w_ref, b_ref, out_ref):
    deg = deg_ref[...]
    dis = jnp.where(deg > 0.0, lax.rsqrt(jnp.where(deg > 0.0, deg, 1.0)), 0.0)
    agg = jnp.concatenate([agg_ref[cc] for cc in range(CHUNKS)], axis=-1)
    r = jnp.dot(h_ref[...], w_ref[...], preferred_element_type=jnp.float32)
    out_ref[...] = jnp.maximum(agg * dis + r + b_ref[...], 0.0)


def _combine(deg, agg, h, w, b):
    n, d_in = h.shape
    return pl.pallas_call(
        _combine_body,
        grid=(n // RB,),
        in_specs=[
            pl.BlockSpec((RB, 1), lambda r: (r, 0)),
            pl.BlockSpec((CHUNKS, RB, CW), lambda r: (0, r, 0)),
            pl.BlockSpec((RB, d_in), lambda r: (r, 0)),
            pl.BlockSpec((d_in, CHUNKS * CW), lambda r: (0, 0)),
            pl.BlockSpec((1, CHUNKS * CW), lambda r: (0, 0)),
        ],
        out_specs=pl.BlockSpec((RB, CHUNKS * CW), lambda r: (r, 0)),
        out_shape=jax.ShapeDtypeStruct((n, CHUNKS * CW), jnp.float32),
    )(deg, agg, h, w, b)


def _fc_body(h_ref, w_ref, b_ref, out_ref):
    out_ref[...] = (
        jnp.dot(h_ref[...], w_ref[...], preferred_element_type=jnp.float32)
        + b_ref[...]
    )


def _fc(h2, w, b):
    m, kdim = h2.shape
    d_out = w.shape[1]
    rb = 400
    return pl.pallas_call(
        _fc_body,
        grid=(m // rb,),
        in_specs=[
            pl.BlockSpec((rb, kdim), lambda r: (r, 0)),
            pl.BlockSpec((kdim, d_out), lambda r: (0, 0)),
            pl.BlockSpec((1, d_out), lambda r: (0, 0)),
        ],
        out_specs=pl.BlockSpec((rb, d_out), lambda r: (r, 0)),
        out_shape=jax.ShapeDtypeStruct((m, d_out), jnp.float32),
    )(h2, w, b)


def _sc_degree(col16, zdeg, ones):
    mesh = plsc.VectorSubcoreMesh(core_axis_name="c", subcore_axis_name="s")

    @functools.partial(
        pl.kernel,
        out_type=jax.ShapeDtypeStruct((NPAD, CW), jnp.float32),
        mesh=mesh,
        scratch_types=[
            pltpu.VMEM_SHARED((NPAD, CW), jnp.float32),
            pltpu.VMEM((NB, K), jnp.int32),
            pltpu.VMEM((K, CW), jnp.float32),
        ],
    )
    def k(col_hbm, z_hbm, ones_hbm, out_hbm, accd, colv, onesv):
        c = lax.axis_index("c")
        s = lax.axis_index("s")
        pltpu.sync_copy(col_hbm.at[s], colv)
        pltpu.sync_copy(ones_hbm, onesv)
        pltpu.sync_copy(z_hbm, accd.at[pl.ds(s * NPT, NPT)])
        plsc.subcore_barrier()

        def body(b, carry):
            pltpu.sync_copy(onesv, accd.at[colv.at[b]], add=True)
            return carry

        lax.fori_loop(0, NB, body, 0)
        plsc.subcore_barrier()

        @pl.when(c == 0)
        def _():
            pltpu.sync_copy(accd.at[pl.ds(s * NPT, NPT)],
                            out_hbm.at[pl.ds(s * NPT, NPT)])

    return k(col16, zdeg, ones)


G = 4
NG = NB // G


def _sc_edge_pass(t_flat, row16, col16, zrows):
    mesh = plsc.VectorSubcoreMesh(core_axis_name="c", subcore_axis_name="s")

    @functools.partial(
        pl.kernel,
        out_type=jax.ShapeDtypeStruct((CHUNKS * NPAD, CW), jnp.float32),
        mesh=mesh,
        scratch_types=[
            pltpu.VMEM_SHARED((NPAD, CW), jnp.float32),
            pltpu.VMEM((NB, K), jnp.int32),
            pltpu.VMEM((2 * G, K), jnp.int32),
            pltpu.VMEM((K, CW), jnp.float32),
            pltpu.VMEM((K, CW), jnp.float32),
            pltpu.SemaphoreType.DMA,
            pltpu.SemaphoreType.DMA,
            pltpu.SemaphoreType.DMA,
        ],
    )
    def k(t_hbm, row_hbm, col_hbm, z_hbm, out_hbm,
          acc, rowv, cg, g0, g1, csem, ss0, ss1):
        c = lax.axis_index("c")
        s = lax.axis_index("s")
        gbufs = (g0, g1)
        ssems = (ss0, ss1)
        pltpu.sync_copy(row_hbm.at[s], rowv)
        for p in range(2):
            chunk = c * 2 + p
            off = chunk * NPAD
            delta = (c * 2) * NPAD if p == 0 else NPAD
            pltpu.sync_copy(z_hbm, acc.at[pl.ds(s * NPT, NPT)])

            def adj_body(b, carry):
                for j in range(K // 16):
                    sl = pl.ds(j * 16, 16)
                    rowv[b, sl] = rowv[b, sl] + delta
                return carry

            lax.fori_loop(0, NB, adj_body, 0)
            plsc.subcore_barrier()

            pltpu.async_copy(col_hbm.at[s, pl.ds(0, G)],
                             cg.at[pl.ds(0, G)], csem)

            def grp_body(g, carry):
                half = lax.rem(g, 2) * G
                ohalf = lax.rem(g + 1, 2) * G
                pltpu.make_async_copy(
                    col_hbm.at[s, pl.ds(g * G, G)],
                    cg.at[pl.ds(half, G)], csem).wait()
                for j in range(G):
                    gb = gbufs[j % 2]
                    sm = ssems[j % 2]
                    idxrow = cg.at[half + j]
                    if j < 2:
                        @pl.when(g > 0)
                        def _():
                            pltpu.make_async_copy(
                                gb, acc.at[idxrow], sm).wait()
                    else:
                        pltpu.make_async_copy(gb, acc.at[idxrow], sm).wait()
                    b = g * G + j
                    pltpu.sync_copy(t_hbm.at[rowv.at[b]], gb)
                    pltpu.async_copy(gb, acc.at[idxrow], sm, add=True)
                    if j == 2:
                        @pl.when(g < NG - 1)
                        def _():
                            pltpu.async_copy(
                                col_hbm.at[s, pl.ds((g + 1) * G, G)],
                                cg.at[pl.ds(ohalf, G)], csem)
                return carry

            lax.fori_loop(0, NG, grp_body, 0)
            pltpu.make_async_copy(g0, acc.at[cg.at[0]], ss0).wait()
            pltpu.make_async_copy(g1, acc.at[cg.at[1]], ss1).wait()
            plsc.subcore_barrier()
            pltpu.sync_copy(acc.at[pl.ds(s * NPT, NPT)],
                            out_hbm.at[pl.ds(off + s * NPT, NPT)])
            plsc.subcore_barrier()

    return k(t_flat, row16, col16, zrows)


def kernel(x, edge_index, w_init, w_root, biases, fc_w, fc_b):
    pad = jnp.full((EPAD - edge_index.shape[1],), N, jnp.int32)
    row16 = jnp.concatenate([edge_index[0], pad]).reshape(NTILES, NB, K)
    col16 = jnp.concatenate([edge_index[1], pad]).reshape(NTILES, NB, K)
    zdeg = jnp.zeros((NPT, CW), jnp.float32)
    ones = jnp.ones((K, CW), jnp.float32)
    zrows = jnp.zeros((NPT, CW), jnp.float32)

    deg = _sc_degree(col16, zdeg, ones)[:, :1]

    h = x
    for i in range(len(w_init)):
        t = _t_chunks(deg, h, w_init[i])
        agg = _sc_edge_pass(t.reshape(CHUNKS * NPAD, CW), row16, col16, zrows)
        h = _combine(deg, agg.reshape(CHUNKS, NPAD, CW), h,
                     w_root[i], biases[i].reshape(1, -1))

    h2 = h.reshape(-1, fc_w.shape[0])
    return _fc(h2, fc_w, fc_b.reshape(1, -1))
